# baseline (speedup 1.0000x reference)
import functools

import numpy as np
import jax
import jax.numpy as jnp
from jax import lax
from jax.experimental import pallas as pl
from jax.experimental.pallas import tpu as pltpu
from jax.experimental.pallas import tpu_sc as plsc

NC, NS = 2, 16
NW = NC * NS
D = 128
B, S, K = 4096, 200, 50
BT = 1024
BSC = B - BT
BB = BSC // NW
NBUF = 4
NR = BB // NBUF


def _make_idx():
    b = np.arange(BT, B, dtype=np.int64)[:, None]
    k = np.arange(K, dtype=np.int64)[None, :]
    idx = b * S + 4 * k
    return idx.reshape(NW, BB, K).astype(np.int32)


_IDX = _make_idx()

_mesh = plsc.VectorSubcoreMesh(core_axis_name="c", subcore_axis_name="s")


@functools.partial(
    pl.kernel,
    out_type=jax.ShapeDtypeStruct((BSC, K, D), jnp.float32),
    mesh=_mesh,
    scratch_types=[
        pltpu.VMEM((BB, K), jnp.int32),
        [pltpu.VMEM((K, D), jnp.float32)] * NBUF,
        [pltpu.SemaphoreType.DMA] * NBUF,
        [pltpu.SemaphoreType.DMA] * NBUF,
    ],
)
def _gather_sc(x_hbm, idx_hbm, out_hbm, idx_v, bufs, gsems, ssems):
    c = lax.axis_index("c")
    s = lax.axis_index("s")
    wid = c * NS + s
    base = wid * BB
    pltpu.sync_copy(idx_hbm.at[wid], idx_v)

    for b in range(NBUF):
        pltpu.async_copy(x_hbm.at[idx_v.at[b]], bufs[b], gsems[b])

    @pl.loop(0, NR)
    def _round(r):
        for b in range(NBUF):
            i = r * NBUF + b
            pltpu.make_async_copy(x_hbm.at[idx_v.at[i]], bufs[b], gsems[b]).wait()
            st = pltpu.async_copy(bufs[b], out_hbm.at[base + i], ssems[b])

            @pl.when(r < NR - 1)
            def _refill():
                st.wait()
                pltpu.async_copy(x_hbm.at[idx_v.at[i + NBUF]], bufs[b], gsems[b])

    for b in range(NBUF):
        i = (NR - 1) * NBUF + b
        pltpu.make_async_copy(bufs[b], out_hbm.at[base + i], ssems[b]).wait()


def _tc_body(x_ref, o_ref, sem):
    for k in range(K):
        pltpu.async_copy(
            x_ref.at[pl.ds(0, BT), 4 * k, :], o_ref.at[pl.ds(0, BT), k, :], sem
        )
    for k in range(K):
        pltpu.make_async_copy(
            x_ref.at[pl.ds(0, BT), 4 * k, :], o_ref.at[pl.ds(0, BT), k, :], sem
        ).wait()


_gather_tc = pl.pallas_call(
    _tc_body,
    out_shape=jax.ShapeDtypeStruct((BT, K, D), jnp.float32),
    in_specs=[pl.BlockSpec(memory_space=pltpu.MemorySpace.HBM)],
    out_specs=pl.BlockSpec(memory_space=pltpu.MemorySpace.HBM),
    scratch_shapes=[pltpu.SemaphoreType.DMA],
)


def kernel(x):
    x2 = x.reshape(B * S, D)
    sc_out = _gather_sc(x2, _IDX)
    tc_out = _gather_tc(x)
    return jnp.concatenate([tc_out, sc_out], axis=0)


# speedup vs baseline: 5.9298x; 5.9298x over previous
import functools

import numpy as np
import jax
import jax.numpy as jnp
from jax import lax
from jax.experimental import pallas as pl
from jax.experimental.pallas import tpu as pltpu
from jax.experimental.pallas import tpu_sc as plsc

NC, NS = 2, 16
NW = NC * NS
D = 128
B, S, K = 4096, 200, 50
BB = B // NW
M = 8
P = 4
NR = BB // M


def _make_idx():
    b = np.arange(B, dtype=np.int64)[:, None]
    k = np.arange(K, dtype=np.int64)[None, :]
    idx = b * S + 4 * k
    return idx.reshape(NW, BB, K).astype(np.int32)


_IDX = _make_idx()

_mesh = plsc.VectorSubcoreMesh(core_axis_name="c", subcore_axis_name="s")


@functools.partial(
    pl.kernel,
    out_type=jax.ShapeDtypeStruct((B, K, D), jnp.float32),
    mesh=_mesh,
    scratch_types=[
        pltpu.VMEM((BB, K), jnp.int32),
        [pltpu.VMEM((K, D), jnp.float32)] * M,
        [pltpu.SemaphoreType.DMA] * M,
        [pltpu.SemaphoreType.DMA] * M,
    ],
)
def _gather_sc(x_hbm, idx_hbm, out_hbm, idx_v, bufs, gsems, ssems):
    c = lax.axis_index("c")
    s = lax.axis_index("s")
    wid = c * NS + s
    base = wid * BB
    pltpu.sync_copy(idx_hbm.at[wid], idx_v)

    for b in range(P):
        pltpu.async_copy(x_hbm.at[idx_v.at[b]], bufs[b], gsems[b])

    @pl.loop(0, NR)
    def _round(r):
        for b in range(M):
            j = r * M + b
            pltpu.make_async_copy(x_hbm.at[idx_v.at[j]], bufs[b], gsems[b]).wait()
            pltpu.async_copy(bufs[b], out_hbm.at[base + j], ssems[b])

            bs = (b - P) % M

            @pl.when(j >= P)
            def _drain():
                pltpu.make_async_copy(
                    bufs[bs], out_hbm.at[base + j - P], ssems[bs]
                ).wait()

            bn = (b + P) % M

            @pl.when(j + P < BB)
            def _refill():
                pltpu.async_copy(x_hbm.at[idx_v.at[j + P]], bufs[bn], gsems[bn])

    for b in range(P):
        j = BB - P + b
        pltpu.make_async_copy(
            bufs[j % M], out_hbm.at[base + j], ssems[j % M]
        ).wait()


def kernel(x):
    x2 = x.reshape(B * S, D)
    return _gather_sc(x2, _IDX)


# speedup vs baseline: 5.9369x; 1.0012x over previous
import functools

import numpy as np
import jax
import jax.numpy as jnp
from jax import lax
from jax.experimental import pallas as pl
from jax.experimental.pallas import tpu as pltpu
from jax.experimental.pallas import tpu_sc as plsc

NC, NS = 2, 16
NW = NC * NS
D = 128
B, S, K = 4096, 200, 50
BB = B // NW
M = 8
P = 4
NR = BB // M


def _make_idx():
    b = np.arange(B, dtype=np.int64)[:, None]
    k = np.arange(K, dtype=np.int64)[None, :]
    idx = b * S + 4 * k
    return idx.reshape(NW, BB, K).astype(np.int32)


_IDX = _make_idx()

_mesh = plsc.VectorSubcoreMesh(core_axis_name="c", subcore_axis_name="s")


@functools.partial(
    pl.kernel,
    out_type=jax.ShapeDtypeStruct((B, K, D), jnp.float32),
    mesh=_mesh,
    compiler_params=pltpu.CompilerParams(use_tc_tiling_on_sc=True),
    scratch_types=[
        pltpu.VMEM((BB, K), jnp.int32),
        [pltpu.VMEM((K, D), jnp.float32)] * M,
        [pltpu.SemaphoreType.DMA] * M,
        [pltpu.SemaphoreType.DMA] * M,
    ],
)
def _gather_sc(x_hbm, idx_hbm, out_hbm, idx_v, bufs, gsems, ssems):
    c = lax.axis_index("c")
    s = lax.axis_index("s")
    wid = c * NS + s
    base = wid * BB
    pltpu.sync_copy(idx_hbm.at[wid], idx_v)

    for b in range(P):
        pltpu.async_copy(x_hbm.at[idx_v.at[b]], bufs[b], gsems[b])

    @pl.loop(0, NR)
    def _round(r):
        for b in range(M):
            j = r * M + b
            pltpu.make_async_copy(x_hbm.at[idx_v.at[j]], bufs[b], gsems[b]).wait()
            pltpu.async_copy(bufs[b], out_hbm.at[base + j], ssems[b])

            bs = (b - P) % M

            @pl.when(j >= P)
            def _drain():
                pltpu.make_async_copy(
                    bufs[bs], out_hbm.at[base + j - P], ssems[bs]
                ).wait()

            bn = (b + P) % M

            @pl.when(j + P < BB)
            def _refill():
                pltpu.async_copy(x_hbm.at[idx_v.at[j + P]], bufs[bn], gsems[bn])

    for b in range(P):
        j = BB - P + b
        pltpu.make_async_copy(
            bufs[j % M], out_hbm.at[base + j], ssems[j % M]
        ).wait()


def kernel(x):
    x2 = x.reshape(B * S, D)
    return _gather_sc(x2, _IDX)


# speedup vs baseline: 5.9376x; 1.0001x over previous
import functools

import numpy as np
import jax
import jax.numpy as jnp
from jax import lax
from jax.experimental import pallas as pl
from jax.experimental.pallas import tpu as pltpu
from jax.experimental.pallas import tpu_sc as plsc

NC, NS = 2, 16
NW = NC * NS
D = 128
B, S, K = 4096, 200, 50
BB = B // NW
M = 8
P = 4
NR = BB // M


def _make_idx():
    b = np.arange(B, dtype=np.int64)[:, None]
    k = np.arange(K, dtype=np.int64)[None, :]
    idx = b * S + 4 * k
    return idx.reshape(NW, BB, K).astype(np.int32)


_IDX = _make_idx()

_mesh = plsc.VectorSubcoreMesh(core_axis_name="c", subcore_axis_name="s")


@functools.partial(
    pl.kernel,
    out_type=jax.ShapeDtypeStruct((B, K, D), jnp.float32),
    mesh=_mesh,
    compiler_params=pltpu.CompilerParams(
        use_tc_tiling_on_sc=True, needs_layout_passes=True
    ),
    scratch_types=[
        pltpu.VMEM((BB, K), jnp.int32),
        [pltpu.VMEM((K, D), jnp.float32)] * M,
        [pltpu.SemaphoreType.DMA] * M,
        [pltpu.SemaphoreType.DMA] * M,
    ],
)
def _gather_sc(x_hbm, idx_hbm, out_hbm, idx_v, bufs, gsems, ssems):
    c = lax.axis_index("c")
    s = lax.axis_index("s")
    wid = c * NS + s
    base = wid * BB
    pltpu.sync_copy(idx_hbm.at[wid], idx_v)

    for b in range(P):
        pltpu.async_copy(x_hbm.at[idx_v.at[b]], bufs[b], gsems[b])

    @pl.loop(0, NR)
    def _round(r):
        for b in range(M):
            j = r * M + b
            pltpu.make_async_copy(x_hbm.at[idx_v.at[j]], bufs[b], gsems[b]).wait()
            pltpu.async_copy(bufs[b], out_hbm.at[base + j], ssems[b])

            bs = (b - P) % M

            @pl.when(j >= P)
            def _drain():
                pltpu.make_async_copy(
                    bufs[bs], out_hbm.at[base + j - P], ssems[bs]
                ).wait()

            bn = (b + P) % M

            @pl.when(j + P < BB)
            def _refill():
                pltpu.async_copy(x_hbm.at[idx_v.at[j + P]], bufs[bn], gsems[bn])

    for b in range(P):
        j = BB - P + b
        pltpu.make_async_copy(
            bufs[j % M], out_hbm.at[base + j], ssems[j % M]
        ).wait()


def kernel(x):
    x2 = x.reshape(B * S, D)
    return _gather_sc(x2, _IDX)
